# baseline (speedup 1.0000x reference)
import functools

import jax
import jax.numpy as jnp
from jax import lax
from jax.experimental import pallas as pl
from jax.experimental.pallas import tpu as pltpu
from jax.experimental.pallas import tpu_sc as plsc

_PATCH = 16
_STRIDE = 8


def kernel(series):
    batch = series.shape[:-1]
    seq_len = series.shape[-1]
    rows = 1
    for d in batch:
        rows *= d
    n_patches = (seq_len - _PATCH) // _STRIDE + 1
    out_w = n_patches * _PATCH

    x = series.reshape(rows, seq_len)

    num_workers = 32
    rows_per_w = rows // num_workers
    c_rows = 4
    n_steps = rows_per_w // c_rows

    mesh = plsc.VectorSubcoreMesh(core_axis_name="c", subcore_axis_name="s")

    @functools.partial(
        pl.kernel,
        out_type=jax.ShapeDtypeStruct((rows, out_w), jnp.float32),
        mesh=mesh,
        scratch_types=[
            pltpu.VMEM((2, c_rows, seq_len), jnp.float32),
            pltpu.VMEM((2, c_rows, out_w), jnp.float32),
            pltpu.SemaphoreType.DMA((2,)),
            pltpu.SemaphoreType.DMA((2,)),
        ],
        compiler_params=pltpu.CompilerParams(use_tc_tiling_on_sc=False),
    )
    def patcher(in_hbm, out_hbm, ibuf, obuf, gsem, wsem):
        wid = lax.axis_index("s") * 2 + lax.axis_index("c")
        base = wid * rows_per_w

        def start_gather(step, slot):
            r0 = base + step * c_rows
            cp = pltpu.make_async_copy(
                in_hbm.at[pl.ds(r0, c_rows)], ibuf.at[slot], gsem.at[slot]
            )
            cp.start()
            return cp

        def make_write(step, slot):
            r0 = base + step * c_rows
            return pltpu.make_async_copy(
                obuf.at[slot], out_hbm.at[pl.ds(r0, c_rows)], wsem.at[slot]
            )

        def repack(slot):
            for r in range(c_rows):
                src = ibuf.at[slot, r]
                dst = obuf.at[slot, r]

                @plsc.parallel_loop(0, n_patches, 1, unroll=8)
                def _(p):
                    dst[pl.ds(p * _PATCH, _PATCH)] = src[pl.ds(p * _STRIDE, _PATCH)]

        writes = [None, None]
        g = start_gather(0, 0)
        for step in range(n_steps):
            slot = step % 2
            nslot = (step + 1) % 2
            g.wait()
            if step + 1 < n_steps:
                g = start_gather(step + 1, nslot)
            if writes[slot] is not None:
                writes[slot].wait()
            repack(slot)
            w = make_write(step, slot)
            w.start()
            writes[slot] = w
        writes[(n_steps - 2) % 2].wait()
        writes[(n_steps - 1) % 2].wait()

    out = patcher(x)
    return out.reshape(*batch, n_patches, _PATCH)


# speedup vs baseline: 1.0775x; 1.0775x over previous
import functools

import jax
import jax.numpy as jnp
from jax import lax
from jax.experimental import pallas as pl
from jax.experimental.pallas import tpu as pltpu
from jax.experimental.pallas import tpu_sc as plsc

_PATCH = 16
_STRIDE = 8


def _sc_patch_rows(x, rows, seq_len, n_patches):
    out_w = n_patches * _PATCH

    num_workers = 32
    rows_per_w = rows // num_workers
    c_rows = 4
    n_steps = rows_per_w // c_rows

    mesh = plsc.VectorSubcoreMesh(core_axis_name="c", subcore_axis_name="s")

    @functools.partial(
        pl.kernel,
        out_type=jax.ShapeDtypeStruct((rows, out_w), jnp.float32),
        mesh=mesh,
        scratch_types=[
            pltpu.VMEM((2, c_rows, seq_len), jnp.float32),
            pltpu.VMEM((2, c_rows, out_w), jnp.float32),
            pltpu.SemaphoreType.DMA((2,)),
            pltpu.SemaphoreType.DMA((2,)),
        ],
        compiler_params=pltpu.CompilerParams(use_tc_tiling_on_sc=False),
    )
    def patcher(in_hbm, out_hbm, ibuf, obuf, gsem, wsem):
        wid = lax.axis_index("s") * 2 + lax.axis_index("c")
        base = wid * rows_per_w

        def start_gather(step, slot):
            r0 = base + step * c_rows
            cp = pltpu.make_async_copy(
                in_hbm.at[pl.ds(r0, c_rows)], ibuf.at[slot], gsem.at[slot]
            )
            cp.start()
            return cp

        def make_write(step, slot):
            r0 = base + step * c_rows
            return pltpu.make_async_copy(
                obuf.at[slot], out_hbm.at[pl.ds(r0, c_rows)], wsem.at[slot]
            )

        def repack(slot):
            for r in range(c_rows):
                src = ibuf.at[slot, r]
                dst = obuf.at[slot, r]

                @plsc.parallel_loop(0, n_patches, 1, unroll=8)
                def _(p):
                    dst[pl.ds(p * _PATCH, _PATCH)] = src[pl.ds(p * _STRIDE, _PATCH)]

        writes = [None, None]
        g = start_gather(0, 0)
        for step in range(n_steps):
            slot = step % 2
            nslot = (step + 1) % 2
            g.wait()
            if step + 1 < n_steps:
                g = start_gather(step + 1, nslot)
            if writes[slot] is not None:
                writes[slot].wait()
            repack(slot)
            w = make_write(step, slot)
            w.start()
            writes[slot] = w
        writes[(n_steps - 2) % 2].wait()
        writes[(n_steps - 1) % 2].wait()

    return patcher(x)


def kernel(series):
    batch = series.shape[:-1]
    seq_len = series.shape[-1]
    rows = 1
    for d in batch:
        rows *= d
    n_patches = (seq_len - _PATCH) // _STRIDE + 1
    out_w = n_patches * _PATCH

    x = series.reshape(rows, seq_len)
    flat = _sc_patch_rows(x, rows, seq_len, n_patches)

    blk = rows // (batch[0] * batch[1])

    def finisher(x_ref, o_ref):
        o_ref[...] = x_ref[...].reshape(1, 1, blk, n_patches, _PATCH)

    b1 = batch[1]
    out = pl.pallas_call(
        finisher,
        grid=(batch[0], batch[1]),
        in_specs=[pl.BlockSpec((blk, out_w), lambda i, j: (i * b1 + j, 0))],
        out_specs=pl.BlockSpec(
            (1, 1, blk, n_patches, _PATCH), lambda i, j: (i, j, 0, 0, 0)
        ),
        out_shape=jax.ShapeDtypeStruct((*batch, n_patches, _PATCH), jnp.float32),
    )(flat)
    return out


# speedup vs baseline: 3.1662x; 2.9384x over previous
import functools

import jax
import jax.numpy as jnp
from jax import lax
from jax.experimental import pallas as pl
from jax.experimental.pallas import tpu as pltpu
from jax.experimental.pallas import tpu_sc as plsc

_PATCH = 16
_STRIDE = 8


def kernel(series):
    batch = series.shape[:-1]
    seq_len = series.shape[-1]
    rows = 1
    for d in batch:
        rows *= d
    n_patches = (seq_len - _PATCH) // _STRIDE + 1

    x = series.reshape(rows, seq_len)

    num_workers = 32
    rows_per_w = rows // num_workers
    c_rows = 4
    n_steps = rows_per_w // c_rows
    n_iters = _PATCH * 32

    mesh = plsc.VectorSubcoreMesh(core_axis_name="c", subcore_axis_name="s")

    @functools.partial(
        pl.kernel,
        out_type=jax.ShapeDtypeStruct((rows, _PATCH, n_patches), jnp.float32),
        mesh=mesh,
        scratch_types=[
            pltpu.VMEM((2, c_rows, seq_len), jnp.float32),
            pltpu.VMEM((2, c_rows, _PATCH, n_patches), jnp.float32),
            pltpu.SemaphoreType.DMA((2,)),
            pltpu.SemaphoreType.DMA((2,)),
        ],
        compiler_params=pltpu.CompilerParams(
            use_tc_tiling_on_sc=False, needs_layout_passes=False
        ),
    )
    def patcher(in_hbm, out_hbm, ibuf, obuf, gsem, wsem):
        wid = lax.axis_index("s") * 2 + lax.axis_index("c")
        base = wid * rows_per_w
        iota8 = jax.lax.iota(jnp.int32, _PATCH) * _STRIDE

        def start_gather(step, slot):
            r0 = base + step * c_rows
            cp = pltpu.make_async_copy(
                in_hbm.at[pl.ds(r0, c_rows)], ibuf.at[slot], gsem.at[slot]
            )
            cp.start()
            return cp

        def make_write(step, slot):
            r0 = base + step * c_rows
            return pltpu.make_async_copy(
                obuf.at[slot], out_hbm.at[pl.ds(r0, c_rows)], wsem.at[slot]
            )

        def repack(slot):
            for r in range(c_rows):
                src = ibuf.at[slot, r]
                dst = obuf.at[slot, r]

                @plsc.parallel_loop(0, n_iters, 1, unroll=4)
                def _(k):
                    t = k >> 5
                    i = k & 31
                    p0 = i << 4

                    @pl.when(i < 31)
                    def _():
                        idx = iota8 + (p0 * _STRIDE + t)
                        v = plsc.load_gather(src, [idx])
                        dst[t, pl.ds(p0, _PATCH)] = v

                @plsc.parallel_loop(0, _PATCH, 1)
                def _(t):
                    idx = jnp.minimum(iota8 + (496 * _STRIDE + t), seq_len - 1)
                    v = plsc.load_gather(src, [idx])
                    tvec = jnp.full((_PATCH,), 0, jnp.int32) + t
                    pvec = jax.lax.iota(jnp.int32, _PATCH) + 496
                    msk = jax.lax.iota(jnp.int32, _PATCH) < (_PATCH - 1)
                    plsc.store_scatter(dst, [tvec, pvec], v, mask=msk)

        writes = [None, None]
        g = start_gather(0, 0)
        for step in range(n_steps):
            slot = step % 2
            nslot = (step + 1) % 2
            g.wait()
            if step + 1 < n_steps:
                g = start_gather(step + 1, nslot)
            if writes[slot] is not None:
                writes[slot].wait()
            repack(slot)
            w = make_write(step, slot)
            w.start()
            writes[slot] = w
        writes[(n_steps - 2) % 2].wait()
        writes[(n_steps - 1) % 2].wait()

    out_t = patcher(x)
    out_t = out_t.reshape(*batch, _PATCH, n_patches)
    return jnp.swapaxes(out_t, -1, -2)


# speedup vs baseline: 4.3667x; 1.3792x over previous
import functools

import jax
import jax.numpy as jnp
from jax import lax
from jax.experimental import pallas as pl
from jax.experimental.pallas import tpu as pltpu
from jax.experimental.pallas import tpu_sc as plsc

_PATCH = 16
_STRIDE = 8


def kernel(series):
    batch = series.shape[:-1]
    seq_len = series.shape[-1]
    rows = 1
    for d in batch:
        rows *= d
    n_patches = (seq_len - _PATCH) // _STRIDE + 1

    x = series.reshape(rows, seq_len)

    num_workers = 32
    rows_per_w = rows // num_workers
    c_rows = 4
    n_steps = rows_per_w // c_rows
    n_iters = _PATCH * 31

    mesh = plsc.VectorSubcoreMesh(core_axis_name="c", subcore_axis_name="s")

    @functools.partial(
        pl.kernel,
        out_type=jax.ShapeDtypeStruct((rows, _PATCH, n_patches), jnp.float32),
        mesh=mesh,
        scratch_types=[
            pltpu.VMEM((2, c_rows, seq_len), jnp.float32),
            pltpu.VMEM((2, c_rows, _PATCH, n_patches), jnp.float32),
            pltpu.SemaphoreType.DMA((2,)),
            pltpu.SemaphoreType.DMA((2,)),
        ],
        compiler_params=pltpu.CompilerParams(
            use_tc_tiling_on_sc=False, needs_layout_passes=False
        ),
    )
    def patcher(in_hbm, out_hbm, ibuf, obuf, gsem, wsem):
        wid = lax.axis_index("s") * 2 + lax.axis_index("c")
        base = wid * rows_per_w
        iota8 = jax.lax.iota(jnp.int32, _PATCH) * _STRIDE

        def start_gather(step, slot):
            r0 = base + step * c_rows
            cp = pltpu.make_async_copy(
                in_hbm.at[pl.ds(r0, c_rows)], ibuf.at[slot], gsem.at[slot]
            )
            cp.start()
            return cp

        def make_write(step, slot):
            r0 = base + step * c_rows
            return pltpu.make_async_copy(
                obuf.at[slot], out_hbm.at[pl.ds(r0, c_rows)], wsem.at[slot]
            )

        def repack(slot):
            for r in range(c_rows):
                src = ibuf.at[slot, r]
                dst = obuf.at[slot, r]

                @plsc.parallel_loop(0, n_iters, 1, unroll=8)
                def _(k):
                    t = k & 15
                    i = k >> 4
                    idx = iota8 + ((i << 7) + t)
                    v = plsc.load_gather(src, [idx])
                    dst[t, pl.ds(i << 4, _PATCH)] = v

                @plsc.parallel_loop(0, _PATCH, 1)
                def _(t):
                    idx = jnp.minimum(iota8 + (496 * _STRIDE + t), seq_len - 1)
                    v = plsc.load_gather(src, [idx])
                    tvec = jnp.full((_PATCH,), 0, jnp.int32) + t
                    pvec = jax.lax.iota(jnp.int32, _PATCH) + 496
                    msk = jax.lax.iota(jnp.int32, _PATCH) < (_PATCH - 1)
                    plsc.store_scatter(dst, [tvec, pvec], v, mask=msk)

        writes = [None, None]
        g = start_gather(0, 0)
        for step in range(n_steps):
            slot = step % 2
            nslot = (step + 1) % 2
            g.wait()
            if step + 1 < n_steps:
                g = start_gather(step + 1, nslot)
            if writes[slot] is not None:
                writes[slot].wait()
            repack(slot)
            w = make_write(step, slot)
            w.start()
            writes[slot] = w
        writes[(n_steps - 2) % 2].wait()
        writes[(n_steps - 1) % 2].wait()

    out_t = patcher(x)
    out_t = out_t.reshape(*batch, _PATCH, n_patches)
    return jnp.swapaxes(out_t, -1, -2)
